# baseline (speedup 1.0000x reference)
import functools

import jax
import jax.numpy as jnp
from jax import lax
from jax.experimental import pallas as pl
from jax.experimental.pallas import tpu as pltpu
from jax.experimental.pallas import tpu_sc as plsc

_P = 256
_T = 8192
_E = 262144
_NS = 16
_EPT = _E // _NS
_QUART = (_P * _T) // 4
_PAD = 16
_TRASH = _QUART
_UNROLL = 8


def _count_body(t2p_src, t2p_dst, p2t_src, p2t_dst, cpt_out, ctp_out,
                srcb, dstb, idxb, ones_v, zero_v, shared):
  cid = lax.axis_index("c")
  sid = lax.axis_index("s")
  ebase = sid * _EPT

  def fill(buf, val):
    vec = jnp.full((16,), val, jnp.float32)
    def fb(i, _):
      buf[pl.ds(i * 16, 16)] = vec
      return 0
    lax.fori_loop(0, _EPT // 16, fb, 0)

  fill(ones_v, 1.0)
  fill(zero_v, 0.0)

  def relation(src_h, dst_h, out_h, q_rows, width):
    pltpu.sync_copy(src_h.at[pl.ds(ebase, _EPT)], srcb)
    pltpu.sync_copy(dst_h.at[pl.ds(ebase, _EPT)], dstb)

    for ph in range(2):
      q = cid * 2 + ph
      lo = q * q_rows
      lo_w = lo * width

      for j in range(_QUART // _NS // _EPT):
        pltpu.sync_copy(zero_v,
                        shared.at[pl.ds(sid * (_QUART // _NS) + j * _EPT,
                                        _EPT)])
      @pl.when(sid == 0)
      def _():
        pltpu.sync_copy(zero_v.at[pl.ds(0, _PAD)],
                        shared.at[pl.ds(_QUART, _PAD)])

      def group_body(i, _):
        work = []
        for u in range(_UNROLL):
          off = (i * _UNROLL + u) * 16
          s = srcb[pl.ds(off, 16)]
          d = dstb[pl.ds(off, 16)]
          m = (d >= lo) & (d < lo + q_rows)
          flat = jnp.where(m, d * width + s - lo_w, _TRASH)
          work.append((off, flat))
        for off, flat in work:
          idxb[pl.ds(off, 16)] = flat
        return 0

      lax.fori_loop(0, _EPT // (16 * _UNROLL), group_body, 0)

      plsc.subcore_barrier()
      pltpu.sync_copy(ones_v, shared.at[idxb], add=True)
      plsc.subcore_barrier()

      off = sid * (_QUART // _NS)
      pltpu.sync_copy(shared.at[pl.ds(off, _QUART // _NS)],
                      out_h.at[pl.ds(q * _QUART + off, _QUART // _NS)])
      plsc.subcore_barrier()

  relation(t2p_src, t2p_dst, cpt_out, _P // 4, _T)
  relation(p2t_src, p2t_dst, ctp_out, _T // 4, _P)


_count_kernel = functools.partial(
    pl.kernel,
    out_type=(jax.ShapeDtypeStruct((_P * _T,), jnp.float32),
              jax.ShapeDtypeStruct((_T * _P,), jnp.float32)),
    mesh=plsc.VectorSubcoreMesh(core_axis_name="c", subcore_axis_name="s"),
    scratch_types=[
        pltpu.VMEM((_EPT,), jnp.int32),
        pltpu.VMEM((_EPT,), jnp.int32),
        pltpu.VMEM((_EPT,), jnp.int32),
        pltpu.VMEM((_EPT,), jnp.float32),
        pltpu.VMEM((_EPT,), jnp.float32),
        pltpu.VMEM_SHARED((_QUART + _PAD,), jnp.float32),
    ],
    compiler_params=pltpu.CompilerParams(needs_layout_passes=False,
                                         use_tc_tiling_on_sc=False),
)(_count_body)


def _dense_body(xp_ref, xt_ref, cpt_ref, ctp_ref,
                ws1p_ref, wn1p_ref, b1p_ref, ws1t_ref, wn1t_ref, b1t_ref,
                ws2p_ref, wn2p_ref, b2p_ref, ws2t_ref, wn2t_ref, b2t_ref,
                out_ref):
  f32 = jnp.float32
  cpt = cpt_ref[...]
  ctp = ctp_ref[...]
  xp = xp_ref[...]
  xt = xt_ref[...]

  inv_deg_p = 1.0 / jnp.maximum(jnp.sum(cpt, axis=1, keepdims=True), 1.0)
  inv_deg_t = 1.0 / jnp.maximum(jnp.sum(ctp, axis=1, keepdims=True), 1.0)

  m_p = jnp.dot(cpt, xt, preferred_element_type=f32) * inv_deg_p
  h_p = jnp.dot(xp, ws1p_ref[...], preferred_element_type=f32)
  h_p += jnp.dot(m_p, wn1p_ref[...], preferred_element_type=f32)
  h_p = jnp.maximum(h_p + b1p_ref[...], 0.0)

  m_t = jnp.dot(ctp, xp, preferred_element_type=f32) * inv_deg_t
  h_t = jnp.dot(xt, ws1t_ref[...], preferred_element_type=f32)
  h_t += jnp.dot(m_t, wn1t_ref[...], preferred_element_type=f32)
  h_t = jnp.maximum(h_t + b1t_ref[...], 0.0)

  m_p2 = jnp.dot(cpt, h_t, preferred_element_type=f32) * inv_deg_p
  o_p = jnp.dot(h_p, ws2p_ref[...], preferred_element_type=f32)
  o_p += jnp.dot(m_p2, wn2p_ref[...], preferred_element_type=f32)
  o_p += b2p_ref[...]

  m_t2 = jnp.dot(ctp, h_p, preferred_element_type=f32) * inv_deg_t
  o_t = jnp.dot(h_t, ws2t_ref[...], preferred_element_type=f32)
  o_t += jnp.dot(m_t2, wn2t_ref[...], preferred_element_type=f32)
  o_t += b2t_ref[...]

  score = lax.dot_general(o_p, o_t, (((1,), (1,)), ((), ())),
                          preferred_element_type=f32)
  smax = jnp.max(score, axis=1, keepdims=True)
  e = jnp.exp(score - smax)
  out_ref[...] = e / jnp.sum(e, axis=1, keepdims=True)


def kernel(x_proc, x_task, t2p_src, t2p_dst, p2t_src, p2t_dst, pred_eid,
           W_self1_p, W_neigh1_p, b1_p, W_self1_t, W_neigh1_t, b1_t,
           W_self2_p, W_neigh2_p, b2_p, W_self2_t, W_neigh2_t, b2_t):
  del pred_eid
  cpt_flat, ctp_flat = _count_kernel(t2p_src, t2p_dst, p2t_src, p2t_dst)
  cpt = cpt_flat.reshape(_P, _T)
  ctp = ctp_flat.reshape(_T, _P)

  out = pl.pallas_call(
      _dense_body,
      out_shape=jax.ShapeDtypeStruct((_P, _T), jnp.float32),
      compiler_params=pltpu.CompilerParams(
          vmem_limit_bytes=120 * 1024 * 1024),
  )(x_proc, x_task, cpt, ctp,
    W_self1_p, W_neigh1_p, b1_p.reshape(1, -1),
    W_self1_t, W_neigh1_t, b1_t.reshape(1, -1),
    W_self2_p, W_neigh2_p, b2_p.reshape(1, -1),
    W_self2_t, W_neigh2_t, b2_t.reshape(1, -1))
  return out


# speedup vs baseline: 4.9926x; 4.9926x over previous
import functools

import jax
import jax.numpy as jnp
from jax import lax
from jax.experimental import pallas as pl
from jax.experimental.pallas import tpu as pltpu
from jax.experimental.pallas import tpu_sc as plsc

_P = 256
_T = 8192
_E = 262144
_NW = 32
_CH = 8192
_NCH = _E // _CH
_SLAB = (_P * _T) // _NW


_UNROLL = 16


def _count_body(t2p_src, t2p_dst, p2t_src, p2t_dst, cpt_out, ctp_out,
                srcb, dstb, sems):
  wid = lax.axis_index("s") * 2 + lax.axis_index("c")
  ones = jnp.ones((16,), jnp.float32)
  zeros16 = jnp.zeros((16,), jnp.float32)

  def relation(src_h, dst_h, out_h, rows, width, slab):
    lo = wid * rows
    lo_w = lo * width

    def start(c, b):
      pltpu.async_copy(src_h.at[pl.ds(c * _CH, _CH)], srcb.at[b], sems.at[b, 0])
      pltpu.async_copy(dst_h.at[pl.ds(c * _CH, _CH)], dstb.at[b], sems.at[b, 1])

    def wait(b):
      pltpu.make_async_copy(src_h.at[pl.ds(0, _CH)], srcb.at[b],
                            sems.at[b, 0]).wait()
      pltpu.make_async_copy(dst_h.at[pl.ds(0, _CH)], dstb.at[b],
                            sems.at[b, 1]).wait()

    def zero_body(i, _):
      slab[pl.ds(i * 16, 16)] = zeros16
      return 0

    start(0, 0)
    start(1, 1)
    lax.fori_loop(0, _SLAB // 16, zero_body, 0)

    def chunk_body(c2, _):
      for b in (0, 1):
        c = c2 * 2 + b
        wait(b)

        def group_body(i, _):
          work = []
          for u in range(_UNROLL):
            off = (i * _UNROLL + u) * 16
            s = srcb[b, pl.ds(off, 16)]
            d = dstb[b, pl.ds(off, 16)]
            m = (d >= lo) & (d < lo + rows)
            flat = jnp.where(m, d * width + s - lo_w, 0)
            work.append((flat, m))
          for flat, m in work:
            plsc.addupdate_scatter(slab, [flat], ones, mask=m)
          return 0

        lax.fori_loop(0, _CH // (16 * _UNROLL), group_body, 0)

        @pl.when(c + 2 < _NCH)
        def _():
          start(c + 2, b)
      return 0

    lax.fori_loop(0, _NCH // 2, chunk_body, 0)
    pltpu.sync_copy(slab, out_h.at[pl.ds(wid * _SLAB, _SLAB)])

  pl.run_scoped(
      lambda slab: relation(t2p_src, t2p_dst, cpt_out, _P // _NW, _T, slab),
      pltpu.VMEM((_SLAB,), jnp.float32))
  pl.run_scoped(
      lambda slab: relation(p2t_src, p2t_dst, ctp_out, _T // _NW, _P, slab),
      pltpu.VMEM((_SLAB,), jnp.float32))


_count_kernel = functools.partial(
    pl.kernel,
    out_type=(jax.ShapeDtypeStruct((_P * _T,), jnp.float32),
              jax.ShapeDtypeStruct((_T * _P,), jnp.float32)),
    mesh=plsc.VectorSubcoreMesh(core_axis_name="c", subcore_axis_name="s"),
    scratch_types=[
        pltpu.VMEM((2, _CH), jnp.int32),
        pltpu.VMEM((2, _CH), jnp.int32),
        pltpu.SemaphoreType.DMA((2, 2)),
    ],
    compiler_params=pltpu.CompilerParams(needs_layout_passes=False,
                                         use_tc_tiling_on_sc=False),
)(_count_body)


def _dense_body(xp_ref, xt_ref, cpt_ref, ctp_ref,
                ws1p_ref, wn1p_ref, b1p_ref, ws1t_ref, wn1t_ref, b1t_ref,
                ws2p_ref, wn2p_ref, b2p_ref, ws2t_ref, wn2t_ref, b2t_ref,
                out_ref):
  f32 = jnp.float32
  cpt = cpt_ref[...]
  ctp = ctp_ref[...]
  xp = xp_ref[...]
  xt = xt_ref[...]

  inv_deg_p = 1.0 / jnp.maximum(jnp.sum(cpt, axis=1, keepdims=True), 1.0)
  inv_deg_t = 1.0 / jnp.maximum(jnp.sum(ctp, axis=1, keepdims=True), 1.0)

  m_p = jnp.dot(cpt, xt, preferred_element_type=f32) * inv_deg_p
  h_p = jnp.dot(xp, ws1p_ref[...], preferred_element_type=f32)
  h_p += jnp.dot(m_p, wn1p_ref[...], preferred_element_type=f32)
  h_p = jnp.maximum(h_p + b1p_ref[...], 0.0)

  m_t = jnp.dot(ctp, xp, preferred_element_type=f32) * inv_deg_t
  h_t = jnp.dot(xt, ws1t_ref[...], preferred_element_type=f32)
  h_t += jnp.dot(m_t, wn1t_ref[...], preferred_element_type=f32)
  h_t = jnp.maximum(h_t + b1t_ref[...], 0.0)

  m_p2 = jnp.dot(cpt, h_t, preferred_element_type=f32) * inv_deg_p
  o_p = jnp.dot(h_p, ws2p_ref[...], preferred_element_type=f32)
  o_p += jnp.dot(m_p2, wn2p_ref[...], preferred_element_type=f32)
  o_p += b2p_ref[...]

  m_t2 = jnp.dot(ctp, h_p, preferred_element_type=f32) * inv_deg_t
  o_t = jnp.dot(h_t, ws2t_ref[...], preferred_element_type=f32)
  o_t += jnp.dot(m_t2, wn2t_ref[...], preferred_element_type=f32)
  o_t += b2t_ref[...]

  score = lax.dot_general(o_p, o_t, (((1,), (1,)), ((), ())),
                          preferred_element_type=f32)
  smax = jnp.max(score, axis=1, keepdims=True)
  e = jnp.exp(score - smax)
  out_ref[...] = e / jnp.sum(e, axis=1, keepdims=True)


def kernel(x_proc, x_task, t2p_src, t2p_dst, p2t_src, p2t_dst, pred_eid,
           W_self1_p, W_neigh1_p, b1_p, W_self1_t, W_neigh1_t, b1_t,
           W_self2_p, W_neigh2_p, b2_p, W_self2_t, W_neigh2_t, b2_t):
  del pred_eid
  cpt_flat, ctp_flat = _count_kernel(t2p_src, t2p_dst, p2t_src, p2t_dst)
  cpt = cpt_flat.reshape(_P, _T)
  ctp = ctp_flat.reshape(_T, _P)

  out = pl.pallas_call(
      _dense_body,
      out_shape=jax.ShapeDtypeStruct((_P, _T), jnp.float32),
      compiler_params=pltpu.CompilerParams(
          vmem_limit_bytes=120 * 1024 * 1024),
  )(x_proc, x_task, cpt, ctp,
    W_self1_p, W_neigh1_p, b1_p.reshape(1, -1),
    W_self1_t, W_neigh1_t, b1_t.reshape(1, -1),
    W_self2_p, W_neigh2_p, b2_p.reshape(1, -1),
    W_self2_t, W_neigh2_t, b2_t.reshape(1, -1))
  return out
